# baseline (speedup 1.0000x reference)
import jax
import jax.numpy as jnp
from jax import lax
from jax.experimental import pallas as pl
from jax.experimental.pallas import tpu as pltpu
from jax.experimental.pallas import tpu_sc as plsc

_N_ATOMS = 3_200_000
_N_GRAPHS = 16384
_N_ELEM = 118
_NC = 2
_NS = 16
_NW = _NC * _NS
_CHUNK = _N_ATOMS // _NW
_BLK = 10_000
_NBLK = _CHUNK // _BLK
_NVEC = _BLK // 16
_NROW = _N_ATOMS // 16
_ROWS_PER_TILE = _CHUNK // 16
_GROWS = 640


def _sc_partial_segsum(lin_hbm, z_hbm, b_hbm, out_hbm,
                       table_v, z0_v, z1_v, b0_v, b1_v, i0_v, i1_v, acc_v, sems):
    zbufs = (z0_v, z1_v)
    bbufs = (b0_v, b1_v)
    ibufs = (i0_v, i1_v)
    cid = lax.axis_index("c")
    sid = lax.axis_index("s")
    wid = cid * _NS + sid
    row_base = wid * _ROWS_PER_TILE

    pltpu.sync_copy(lin_hbm, table_v)

    zero16 = jnp.zeros((16,), jnp.float32)

    @plsc.parallel_loop(0, _N_GRAPHS // 16, unroll=8)
    def _(i):
        acc_v[pl.ds(i * 16, 16)] = zero16

    iota = lax.iota(jnp.int32, 16)
    is_last = iota == 15
    not_last = iota < 15
    row_inc = jnp.where(is_last, 1, 0)
    col_shift = jnp.where(is_last, 0, iota + 1)

    def _fill_idx(blk, k):
        ib = ibufs[k]
        first = row_base + blk * _NVEC

        @plsc.parallel_loop(0, _GROWS // 16, unroll=4)
        def _(j):
            ib[pl.ds(j * 16, 16)] = jnp.minimum(first + j * 16 + iota, _NROW - 1)

    def _copies(blk, k):
        return (
            pltpu.make_async_copy(z_hbm.at[ibufs[k]], zbufs[k], sems.at[k, 0]),
            pltpu.make_async_copy(b_hbm.at[ibufs[k]], bbufs[k], sems.at[k, 1]),
        )

    def _start(blk, k):
        _fill_idx(blk, k)
        for c in _copies(blk, k):
            c.start()

    def _wait(blk, k):
        for c in _copies(blk, k):
            c.wait()

    def _compute(k):
        zb = zbufs[k]
        bb = bbufs[k]

        @plsc.parallel_loop(0, _NVEC, unroll=4)
        def _(r):
            z = zb[r]
            b = bb[r]
            bn = plsc.load_gather(bb, [r + row_inc, col_shift])
            v = plsc.load_gather(table_v, [z * 16 + iota])
            c = plsc.cumsum(v)
            m = (b != bn) & not_last
            plsc.addupdate_scatter(acc_v, [b], c, mask=m | is_last)
            plsc.addupdate_scatter(acc_v, [bn], -c, mask=m)

    _start(0, 0)

    def blk_pair(p, _):
        blk = 2 * p
        _start(blk + 1, 1)
        _wait(blk, 0)
        _compute(0)

        @pl.when(blk + 2 < _NBLK)
        def _():
            _start(blk + 2, 0)

        _wait(blk + 1, 1)
        _compute(1)
        return 0

    lax.fori_loop(0, _NBLK // 2, blk_pair, 0)

    if _NBLK % 2 == 1:
        _wait(_NBLK - 1, 0)
        _compute(0)

    pltpu.sync_copy(acc_v, out_hbm.at[wid])


def _merge_body(p_ref, o_ref):
    o_ref[...] = jnp.sum(p_ref[...], axis=0, keepdims=True)


@jax.jit
def kernel(lin_ref, atomic_numbers, batch_ids):
    sc = pl.kernel(
        _sc_partial_segsum,
        out_type=jax.ShapeDtypeStruct((_NW, _N_GRAPHS), jnp.float32),
        mesh=plsc.VectorSubcoreMesh(core_axis_name="c", subcore_axis_name="s"),
        compiler_params=pltpu.CompilerParams(
            needs_layout_passes=False, use_tc_tiling_on_sc=False
        ),
        scratch_types=[
            pltpu.VMEM((_N_ELEM * 16,), jnp.float32),
            pltpu.VMEM((_GROWS, 16), jnp.int32),
            pltpu.VMEM((_GROWS, 16), jnp.int32),
            pltpu.VMEM((_GROWS, 16), jnp.int32),
            pltpu.VMEM((_GROWS, 16), jnp.int32),
            pltpu.VMEM((_GROWS,), jnp.int32),
            pltpu.VMEM((_GROWS,), jnp.int32),
            pltpu.VMEM((_N_GRAPHS,), jnp.float32),
            pltpu.SemaphoreType.DMA((2, 2)),
        ],
    )
    lin_rep = jnp.reshape(
        jnp.broadcast_to(lin_ref[:, None], (_N_ELEM, 16)), (_N_ELEM * 16,)
    )
    z2 = jnp.reshape(atomic_numbers, (_NROW, 16))
    b2 = jnp.reshape(batch_ids, (_NROW, 16))
    partials = sc(lin_rep, z2, b2)
    merged = pl.pallas_call(
        _merge_body,
        out_shape=jax.ShapeDtypeStruct((1, _N_GRAPHS), jnp.float32),
    )(partials)
    return jnp.reshape(merged, (_N_GRAPHS,))


# speedup vs baseline: 1.0087x; 1.0087x over previous
import jax
import jax.numpy as jnp
from jax import lax
from jax.experimental import pallas as pl
from jax.experimental.pallas import tpu as pltpu
from jax.experimental.pallas import tpu_sc as plsc

_N_ATOMS = 3_200_000
_N_GRAPHS = 16384
_N_ELEM = 118
_NC = 2
_NS = 16
_NW = _NC * _NS
_CHUNK = _N_ATOMS // _NW
_BLK = 10_000
_NBLK = _CHUNK // _BLK
_NVEC = _BLK // 16
_NROW = _N_ATOMS // 16
_ROWS_PER_TILE = _CHUNK // 16
_GROWS = 640


def _sc_partial_segsum(lin_hbm, z_hbm, b_hbm, out_hbm,
                       table_v, z0_v, z1_v, b0_v, b1_v, i0_v, i1_v, acc_v, sems):
    zbufs = (z0_v, z1_v)
    bbufs = (b0_v, b1_v)
    ibufs = (i0_v, i1_v)
    cid = lax.axis_index("c")
    sid = lax.axis_index("s")
    wid = cid * _NS + sid
    row_base = wid * _ROWS_PER_TILE
    atom_base = pl.multiple_of(wid * _CHUNK, 8)

    pltpu.sync_copy(lin_hbm, table_v)

    zero16 = jnp.zeros((16,), jnp.float32)

    @plsc.parallel_loop(0, _N_GRAPHS // 16, unroll=8)
    def _(i):
        acc_v[pl.ds(i * 16, 16)] = zero16

    iota = lax.iota(jnp.int32, 16)
    is_last = iota == 15
    not_last = iota < 15
    row_inc = jnp.where(is_last, 1, 0)
    col_shift = jnp.where(is_last, 0, iota + 1)

    def _fill_idx(blk, k):
        ib = ibufs[k]
        first = row_base + blk * _NVEC

        @plsc.parallel_loop(0, _GROWS // 16, unroll=4)
        def _(j):
            ib[pl.ds(j * 16, 16)] = jnp.minimum(first + j * 16 + iota, _NROW - 1)

    def _copies(blk, k):
        boff = pl.multiple_of(atom_base + blk * _BLK, 8)
        return (
            pltpu.make_async_copy(z_hbm.at[ibufs[k]], zbufs[k], sems.at[k, 0]),
            pltpu.make_async_copy(b_hbm.at[pl.ds(boff, _BLK)], bbufs[k].at[pl.ds(0, _BLK)], sems.at[k, 1]),
        )

    def _start(blk, k):
        _fill_idx(blk, k)
        for c in _copies(blk, k):
            c.start()

    def _wait(blk, k):
        for c in _copies(blk, k):
            c.wait()

    def _compute(k):
        zb = zbufs[k]
        bb = bbufs[k]

        @plsc.parallel_loop(0, _NVEC, unroll=4)
        def _(r):
            s = r * 16
            z = zb[r]
            b = bb[pl.ds(s, 16)]
            bn = bb[pl.ds(s + 1, 16)]
            v = plsc.load_gather(table_v, [z * 16 + iota])
            c = plsc.cumsum(v)
            m = (b != bn) & not_last
            plsc.addupdate_scatter(acc_v, [b], c, mask=m | is_last)
            plsc.addupdate_scatter(acc_v, [bn], -c, mask=m)

    _start(0, 0)

    def blk_pair(p, _):
        blk = 2 * p
        _start(blk + 1, 1)
        _wait(blk, 0)
        _compute(0)

        @pl.when(blk + 2 < _NBLK)
        def _():
            _start(blk + 2, 0)

        _wait(blk + 1, 1)
        _compute(1)
        return 0

    lax.fori_loop(0, _NBLK // 2, blk_pair, 0)

    if _NBLK % 2 == 1:
        _wait(_NBLK - 1, 0)
        _compute(0)

    pltpu.sync_copy(acc_v, out_hbm.at[wid])


def _merge_body(p_ref, o_ref):
    o_ref[...] = jnp.sum(p_ref[...], axis=0, keepdims=True)


@jax.jit
def kernel(lin_ref, atomic_numbers, batch_ids):
    sc = pl.kernel(
        _sc_partial_segsum,
        out_type=jax.ShapeDtypeStruct((_NW, _N_GRAPHS), jnp.float32),
        mesh=plsc.VectorSubcoreMesh(core_axis_name="c", subcore_axis_name="s"),
        compiler_params=pltpu.CompilerParams(
            needs_layout_passes=False, use_tc_tiling_on_sc=False
        ),
        scratch_types=[
            pltpu.VMEM((_N_ELEM * 16,), jnp.float32),
            pltpu.VMEM((_GROWS, 16), jnp.int32),
            pltpu.VMEM((_GROWS, 16), jnp.int32),
            pltpu.VMEM((_BLK + 16,), jnp.int32),
            pltpu.VMEM((_BLK + 16,), jnp.int32),
            pltpu.VMEM((_GROWS,), jnp.int32),
            pltpu.VMEM((_GROWS,), jnp.int32),
            pltpu.VMEM((_N_GRAPHS,), jnp.float32),
            pltpu.SemaphoreType.DMA((2, 2)),
        ],
    )
    lin_rep = jnp.reshape(
        jnp.broadcast_to(lin_ref[:, None], (_N_ELEM, 16)), (_N_ELEM * 16,)
    )
    z2 = jnp.reshape(atomic_numbers, (_NROW, 16))
    partials = sc(lin_rep, z2, batch_ids)
    merged = pl.pallas_call(
        _merge_body,
        out_shape=jax.ShapeDtypeStruct((1, _N_GRAPHS), jnp.float32),
    )(partials)
    return jnp.reshape(merged, (_N_GRAPHS,))


# speedup vs baseline: 1.0585x; 1.0494x over previous
import functools

import jax
import jax.numpy as jnp
from jax import lax
from jax.experimental import pallas as pl
from jax.experimental.pallas import tpu as pltpu
from jax.experimental.pallas import tpu_sc as plsc

_N_ATOMS = 3_200_000
_N_GRAPHS = 16384
_N_ELEM = 118
_NC = 2
_NS = 16
_NW = _NC * _NS
_CHUNK = _N_ATOMS // _NW
_BLK = 10_000
_NBLK = _CHUNK // _BLK
_NVEC = _BLK // 16


def _sc_partial_segsum(lin_hbm, z_hbm, b_hbm, out_hbm, table_v, z0_v, z1_v, b0_v, b1_v, acc_v, sems):
    zbufs = (z0_v, z1_v)
    bbufs = (b0_v, b1_v)
    cid = lax.axis_index("c")
    sid = lax.axis_index("s")
    wid = cid * _NS + sid
    base = pl.multiple_of(wid * _CHUNK, 8)

    pltpu.sync_copy(lin_hbm, table_v)

    zero16 = jnp.zeros((16,), jnp.float32)

    @plsc.parallel_loop(0, _N_GRAPHS // 16, unroll=8)
    def _(i):
        acc_v[pl.ds(i * 16, 16)] = zero16

    iota = lax.iota(jnp.int32, 16)
    is_last = iota == 15
    not_last = iota < 15

    def _copies(blk, k):
        off = pl.multiple_of(base + blk * _BLK, 8)
        return (
            pltpu.make_async_copy(z_hbm.at[pl.ds(off, _BLK)], zbufs[k], sems.at[k, 0]),
            pltpu.make_async_copy(b_hbm.at[pl.ds(off, _BLK)], bbufs[k].at[pl.ds(0, _BLK)], sems.at[k, 1]),
        )

    def _start(blk, k):
        for c in _copies(blk, k):
            c.start()

    def _wait(blk, k):
        for c in _copies(blk, k):
            c.wait()

    def _compute(k):
        zb = zbufs[k]
        bb = bbufs[k]

        @plsc.parallel_loop(0, _NVEC, unroll=4)
        def _(i):
            s = i * 16
            z = zb[pl.ds(s, 16)]
            b = bb[pl.ds(s, 16)]
            bn = bb[pl.ds(s + 1, 16)]
            v = plsc.load_gather(table_v, [z])
            c = plsc.cumsum(v)
            m = (b != bn) & not_last
            plsc.addupdate_scatter(acc_v, [b], c, mask=m | is_last)
            plsc.addupdate_scatter(acc_v, [bn], -c, mask=m)

    _start(0, 0)

    def blk_pair(p, _):
        blk = 2 * p
        _start(blk + 1, 1)
        _wait(blk, 0)
        _compute(0)

        @pl.when(blk + 2 < _NBLK)
        def _():
            _start(blk + 2, 0)

        _wait(blk + 1, 1)
        _compute(1)
        return 0

    lax.fori_loop(0, _NBLK // 2, blk_pair, 0)

    if _NBLK % 2 == 1:
        _wait(_NBLK - 1, 0)
        _compute(0)

    pltpu.sync_copy(acc_v, out_hbm.at[wid])


def _merge_body(p_ref, o_ref):
    o_ref[...] = jnp.sum(p_ref[...], axis=0, keepdims=True)


@jax.jit
def kernel(lin_ref, atomic_numbers, batch_ids):
    sc = pl.kernel(
        _sc_partial_segsum,
        out_type=jax.ShapeDtypeStruct((_NW, _N_GRAPHS), jnp.float32),
        mesh=plsc.VectorSubcoreMesh(core_axis_name="c", subcore_axis_name="s"),
        compiler_params=pltpu.CompilerParams(needs_layout_passes=False),
        scratch_types=[
            pltpu.VMEM((_N_ELEM,), jnp.float32),
            pltpu.VMEM((_BLK,), jnp.int32),
            pltpu.VMEM((_BLK,), jnp.int32),
            pltpu.VMEM((_BLK + 16,), jnp.int32),
            pltpu.VMEM((_BLK + 16,), jnp.int32),
            pltpu.VMEM((_N_GRAPHS,), jnp.float32),
            pltpu.SemaphoreType.DMA((2, 2)),
        ],
    )
    partials = sc(lin_ref, atomic_numbers, batch_ids)
    merged = pl.pallas_call(
        _merge_body,
        out_shape=jax.ShapeDtypeStruct((1, _N_GRAPHS), jnp.float32),
    )(partials)
    return jnp.reshape(merged, (_N_GRAPHS,))


# speedup vs baseline: 1.0836x; 1.0237x over previous
import functools

import jax
import jax.numpy as jnp
from jax import lax
from jax.experimental import pallas as pl
from jax.experimental.pallas import tpu as pltpu
from jax.experimental.pallas import tpu_sc as plsc

_N_ATOMS = 3_200_000
_N_GRAPHS = 16384
_N_ELEM = 118
_NC = 2
_NS = 16
_NW = _NC * _NS
_CHUNK = _N_ATOMS // _NW
_BLK = 10_000
_NBLK = _CHUNK // _BLK
_NVEC = _BLK // 16


def _sc_partial_segsum(lin_hbm, z_hbm, b_hbm, out_hbm, table_v, z0_v, z1_v, b0_v, b1_v, acc_v, sems):
    zbufs = (z0_v, z1_v)
    bbufs = (b0_v, b1_v)
    cid = lax.axis_index("c")
    sid = lax.axis_index("s")
    wid = cid * _NS + sid
    base = pl.multiple_of(wid * _CHUNK, 8)

    pltpu.sync_copy(lin_hbm, table_v)

    zero16 = jnp.zeros((16,), jnp.float32)

    @plsc.parallel_loop(0, _N_GRAPHS // 16, unroll=8)
    def _(i):
        acc_v[pl.ds(i * 16, 16)] = zero16

    iota = lax.iota(jnp.int32, 16)
    is_last = iota == 15
    not_last = iota < 15

    def _copies(blk, k):
        off = pl.multiple_of(base + blk * _BLK, 8)
        return (
            pltpu.make_async_copy(z_hbm.at[pl.ds(off, _BLK)], zbufs[k], sems.at[k, 0]),
            pltpu.make_async_copy(b_hbm.at[pl.ds(off, _BLK)], bbufs[k].at[pl.ds(0, _BLK)], sems.at[k, 1]),
        )

    def _start(blk, k):
        for c in _copies(blk, k):
            c.start()

    def _wait(blk, k):
        for c in _copies(blk, k):
            c.wait()

    def _compute(k):
        zb = zbufs[k]
        bb = bbufs[k]

        @plsc.parallel_loop(0, _NVEC, unroll=2)
        def _(i):
            s = i * 16
            z = zb[pl.ds(s, 16)]
            b = bb[pl.ds(s, 16)]
            bn = bb[pl.ds(s + 1, 16)]
            v = plsc.load_gather(table_v, [z * 16 + iota])
            c = plsc.cumsum(v)
            m = (b != bn) & not_last
            plsc.addupdate_scatter(acc_v, [b], c, mask=m | is_last)
            plsc.addupdate_scatter(acc_v, [bn], -c, mask=m)

    _start(0, 0)

    def blk_pair(p, _):
        blk = 2 * p
        _start(blk + 1, 1)
        _wait(blk, 0)
        _compute(0)

        @pl.when(blk + 2 < _NBLK)
        def _():
            _start(blk + 2, 0)

        _wait(blk + 1, 1)
        _compute(1)
        return 0

    lax.fori_loop(0, _NBLK // 2, blk_pair, 0)

    if _NBLK % 2 == 1:
        _wait(_NBLK - 1, 0)
        _compute(0)

    pltpu.sync_copy(acc_v, out_hbm.at[wid])


def _merge_body(p_ref, o_ref):
    o_ref[...] = jnp.sum(p_ref[...], axis=0, keepdims=True)


@jax.jit
def kernel(lin_ref, atomic_numbers, batch_ids):
    sc = pl.kernel(
        _sc_partial_segsum,
        out_type=jax.ShapeDtypeStruct((_NW, _N_GRAPHS), jnp.float32),
        mesh=plsc.VectorSubcoreMesh(core_axis_name="c", subcore_axis_name="s"),
        compiler_params=pltpu.CompilerParams(needs_layout_passes=False),
        scratch_types=[
            pltpu.VMEM((_N_ELEM * 16,), jnp.float32),
            pltpu.VMEM((_BLK,), jnp.int32),
            pltpu.VMEM((_BLK,), jnp.int32),
            pltpu.VMEM((_BLK + 16,), jnp.int32),
            pltpu.VMEM((_BLK + 16,), jnp.int32),
            pltpu.VMEM((_N_GRAPHS,), jnp.float32),
            pltpu.SemaphoreType.DMA((2, 2)),
        ],
    )
    lin_rep = jnp.reshape(
        jnp.broadcast_to(lin_ref[:, None], (_N_ELEM, 16)), (_N_ELEM * 16,)
    )
    partials = sc(lin_rep, atomic_numbers, batch_ids)
    merged = pl.pallas_call(
        _merge_body,
        out_shape=jax.ShapeDtypeStruct((1, _N_GRAPHS), jnp.float32),
    )(partials)
    return jnp.reshape(merged, (_N_GRAPHS,))


# speedup vs baseline: 1.1173x; 1.0311x over previous
import functools

import jax
import jax.numpy as jnp
from jax import lax
from jax.experimental import pallas as pl
from jax.experimental.pallas import tpu as pltpu
from jax.experimental.pallas import tpu_sc as plsc

_N_ATOMS = 3_200_000
_N_GRAPHS = 16384
_N_ELEM = 118
_NC = 2
_NS = 16
_NW = _NC * _NS
_CHUNK = _N_ATOMS // _NW
_BLK = 10_000
_NBLK = _CHUNK // _BLK
_NVEC = _BLK // 16


def _sc_partial_segsum(lin_hbm, z_hbm, b_hbm, out_hbm, table_v, z0_v, z1_v, b0_v, b1_v, acc_v, sems):
    zbufs = (z0_v, z1_v)
    bbufs = (b0_v, b1_v)
    cid = lax.axis_index("c")
    sid = lax.axis_index("s")
    wid = cid * _NS + sid
    base = pl.multiple_of(wid * _CHUNK, 8)

    zero16 = jnp.zeros((16,), jnp.float32)

    iota = lax.iota(jnp.int32, 16)
    is_last = iota == 15
    not_last = iota < 15

    def _copies(blk, k):
        off = pl.multiple_of(base + blk * _BLK, 8)
        return (
            pltpu.make_async_copy(z_hbm.at[pl.ds(off, _BLK)], zbufs[k], sems.at[k, 0]),
            pltpu.make_async_copy(b_hbm.at[pl.ds(off, _BLK)], bbufs[k].at[pl.ds(0, _BLK)], sems.at[k, 1]),
        )

    def _start(blk, k):
        for c in _copies(blk, k):
            c.start()

    def _wait(blk, k):
        for c in _copies(blk, k):
            c.wait()

    def _compute(k):
        zb = zbufs[k]
        bb = bbufs[k]

        @plsc.parallel_loop(0, _NVEC, unroll=2)
        def _(i):
            s = i * 16
            z = zb[pl.ds(s, 16)]
            b = bb[pl.ds(s, 16)]
            bn = bb[pl.ds(s + 1, 16)]
            v = plsc.load_gather(table_v, [z * 16 + iota])
            c = plsc.cumsum(v)
            m = (b != bn) & not_last
            plsc.addupdate_scatter(acc_v, [b], c, mask=m | is_last)
            plsc.addupdate_scatter(acc_v, [bn], -c, mask=m)

    _start(0, 0)
    pltpu.sync_copy(lin_hbm, table_v)

    @plsc.parallel_loop(0, _N_GRAPHS // 16, unroll=8)
    def _(i):
        acc_v[pl.ds(i * 16, 16)] = zero16

    def blk_pair(p, _):
        blk = 2 * p
        _start(blk + 1, 1)
        _wait(blk, 0)
        _compute(0)

        @pl.when(blk + 2 < _NBLK)
        def _():
            _start(blk + 2, 0)

        _wait(blk + 1, 1)
        _compute(1)
        return 0

    lax.fori_loop(0, _NBLK // 2, blk_pair, 0)

    if _NBLK % 2 == 1:
        _wait(_NBLK - 1, 0)
        _compute(0)

    pltpu.sync_copy(acc_v, out_hbm.at[wid])


def _merge_body(p_ref, o_ref):
    o_ref[...] = jnp.sum(p_ref[...], axis=0, keepdims=True)


@jax.jit
def kernel(lin_ref, atomic_numbers, batch_ids):
    sc = pl.kernel(
        _sc_partial_segsum,
        out_type=jax.ShapeDtypeStruct((_NW, _N_GRAPHS), jnp.float32),
        mesh=plsc.VectorSubcoreMesh(core_axis_name="c", subcore_axis_name="s"),
        compiler_params=pltpu.CompilerParams(needs_layout_passes=False),
        scratch_types=[
            pltpu.VMEM((_N_ELEM * 16,), jnp.float32),
            pltpu.VMEM((_BLK,), jnp.int32),
            pltpu.VMEM((_BLK,), jnp.int32),
            pltpu.VMEM((_BLK + 16,), jnp.int32),
            pltpu.VMEM((_BLK + 16,), jnp.int32),
            pltpu.VMEM((_N_GRAPHS,), jnp.float32),
            pltpu.SemaphoreType.DMA((2, 2)),
        ],
    )
    lin_rep = jnp.reshape(
        jnp.broadcast_to(lin_ref[:, None], (_N_ELEM, 16)), (_N_ELEM * 16,)
    )
    partials = sc(lin_rep, atomic_numbers, batch_ids)
    merged = pl.pallas_call(
        _merge_body,
        out_shape=jax.ShapeDtypeStruct((1, _N_GRAPHS), jnp.float32),
    )(partials)
    return jnp.reshape(merged, (_N_GRAPHS,))
